# baseline (speedup 1.0000x reference)
import jax
import jax.numpy as jnp
from jax.experimental import pallas as pl

N = 50000
D = 128


def _mm_body(x_ref, w_ref, o_ref):
    o_ref[...] = jnp.dot(x_ref[...], w_ref[...], preferred_element_type=jnp.float32)


def _mm(x, w):
    return pl.pallas_call(
        _mm_body,
        grid=(50,),
        in_specs=[
            pl.BlockSpec((1000, 128), lambda i: (i, 0)),
            pl.BlockSpec((128, 128), lambda i: (0, 0)),
        ],
        out_specs=pl.BlockSpec((1000, 128), lambda i: (i, 0)),
        out_shape=jax.ShapeDtypeStruct((N, D), jnp.float32),
    )(x, w)


def kernel(h, edge_index_rel0, edge_index_rel1, edge_index_rel2, W0, b0, W1, b1, W2, b2):
    eis = [edge_index_rel0, edge_index_rel1, edge_index_rel2]
    Ws = [W0, W1, W2]
    bs = [b0, b1, b2]
    out = jnp.zeros((N, D), jnp.float32)
    for ei, W, b in zip(eis, Ws, bs):
        src, dst = ei[0], ei[1]
        out_deg = jnp.zeros((N,), jnp.float32).at[src].add(1.0)
        in_deg = jnp.zeros((N,), jnp.float32).at[dst].add(1.0)
        ns = jnp.where(out_deg > 0, jax.lax.rsqrt(jnp.maximum(out_deg, 1e-12)), 0.0)
        nd = jnp.where(in_deg > 0, jax.lax.rsqrt(jnp.maximum(in_deg, 1e-12)), 0.0)
        hfeat = _mm(h * ns[:, None], W)
        msg = jnp.take(hfeat, src, axis=0)
        agg = jnp.zeros((N, D), jnp.float32).at[dst].add(msg)
        out = out + agg * nd[:, None] + b
    return out
